# baseline (speedup 1.0000x reference)
import numpy as np
import jax
import jax.numpy as jnp
from jax import lax
from jax.experimental import pallas as pl
from jax.experimental.pallas import tpu as pltpu
from jax.experimental.pallas import tpu_sc as plsc

MAX_SEQ_LEN = 200
EMBED_DIM = 64
BATCH = 4096
VOCAB = 100000

_N = BATCH * MAX_SEQ_LEN
_S = 128
_NCH = _N // _S
_NC, _NS = 2, 16
_NW = _NC * _NS
_CPW = _NCH // _NW


def _pos_encoding(max_seq_len, wordvec_size):
    pos = np.arange(max_seq_len).reshape(1, -1).T
    i = np.arange(wordvec_size / 2).reshape(1, -1)
    pos_emb = np.empty((max_seq_len, wordvec_size))
    pos_emb[:, 0::2] = np.sin(pos / np.power(10000, 2 * i / wordvec_size))
    pos_emb[:, 1::2] = np.cos(pos / np.power(10000, 2 * i / wordvec_size))
    return pos_emb.astype(np.float32)


_POS = _pos_encoding(MAX_SEQ_LEN, EMBED_DIM)

_TC_PARAMS = pltpu.CompilerParams(dimension_semantics=("parallel",))


def _mask_body(x_ref, o_ref):
    o_ref[...] = (x_ref[...] != 0).astype(jnp.float32)


def _padding_mask(inputs):
    return pl.pallas_call(
        _mask_body,
        out_shape=jax.ShapeDtypeStruct((BATCH, MAX_SEQ_LEN), jnp.float32),
    )(inputs)


def _pack_body(t_ref, o_ref):
    t = t_ref[...]
    o_ref[...] = jnp.concatenate(
        [t, jnp.zeros((t.shape[0], 128 - EMBED_DIM), jnp.float32)], axis=1)


def _tc_pack_table(table):
    blk = 5000
    return pl.pallas_call(
        _pack_body,
        grid=(VOCAB // blk,),
        in_specs=[pl.BlockSpec((blk, EMBED_DIM), lambda i: (i, 0))],
        out_specs=pl.BlockSpec((blk, 128), lambda i: (i, 0)),
        out_shape=jax.ShapeDtypeStruct((VOCAB, 128), jnp.float32),
        compiler_params=_TC_PARAMS,
    )(table)


def _finish_body(g_ref, p_ref, o_ref):
    nb, sl = o_ref.shape[0], o_ref.shape[1]
    x = g_ref[...].reshape(nb, sl, 128)
    o_ref[...] = x[:, :, 0:EMBED_DIM] + p_ref[...][None]


def _tc_finish(gathered, pos, seq_len):
    blk = 64
    return pl.pallas_call(
        _finish_body,
        grid=(BATCH // blk,),
        in_specs=[
            pl.BlockSpec((blk * seq_len, 128), lambda i: (i, 0)),
            pl.BlockSpec((seq_len, EMBED_DIM), lambda i: (0, 0)),
        ],
        out_specs=pl.BlockSpec((blk, seq_len, EMBED_DIM),
                               lambda i: (i, 0, 0)),
        out_shape=jax.ShapeDtypeStruct((BATCH, seq_len, EMBED_DIM),
                                       jnp.float32),
        compiler_params=_TC_PARAMS,
    )(gathered, pos)


def _sc_gather(table128, idx128):
    nch = idx128.shape[0]
    cpw = nch // _NW
    mesh = plsc.VectorSubcoreMesh(core_axis_name="c", subcore_axis_name="s")

    @pl.kernel(
        out_type=jax.ShapeDtypeStruct((nch * _S, 128), jnp.float32),
        mesh=mesh,
        compiler_params=pltpu.CompilerParams(use_tc_tiling_on_sc=True),
        scratch_types=[
            pltpu.VMEM((cpw, _S), jnp.int32),
            pltpu.VMEM((4, _S, 128), jnp.float32),
            pltpu.SemaphoreType.DMA,
            pltpu.SemaphoreType.DMA,
            pltpu.SemaphoreType.DMA,
            pltpu.SemaphoreType.DMA,
            pltpu.SemaphoreType.DMA,
            pltpu.SemaphoreType.DMA,
            pltpu.SemaphoreType.DMA,
            pltpu.SemaphoreType.DMA,
        ],
    )
    def k(table_hbm, idx_hbm, out_hbm, idx_v, rows_v,
          sg0, sg1, sg2, sg3, so0, so1, so2, so3):
        wid = lax.axis_index("s") * _NC + lax.axis_index("c")
        base = wid * cpw
        s_g = (sg0, sg1, sg2, sg3)
        s_o = (so0, so1, so2, so3)

        pltpu.sync_copy(idx_hbm.at[pl.ds(base, cpw)], idx_v)

        def gather(c, b):
            return pltpu.make_async_copy(
                table_hbm.at[idx_v.at[c]], rows_v.at[b], s_g[b])

        def out_copy(c, b):
            return pltpu.make_async_copy(
                rows_v.at[b],
                out_hbm.at[pl.ds((base + c) * _S, _S)], s_o[b])

        gather(0, 0).start()
        gather(1, 1).start()

        @pl.loop(0, cpw, step=4)
        def _(cc):
            for j in range(4):
                c = cc + j
                b = j
                nb = (j + 2) % 4

                gather(c, b).wait()
                out_copy(c, b).start()

                @pl.when(c >= 2)
                def _():
                    out_copy(c - 2, nb).wait()

                @pl.when(c + 2 < cpw)
                def _():
                    gather(c + 2, nb).start()

        out_copy(cpw - 2, 2).wait()
        out_copy(cpw - 1, 3).wait()

    return k(table128, idx128)


_NPART = 5
_SPART = MAX_SEQ_LEN // _NPART


def kernel(inputs, embed_table):
    mask = _padding_mask(inputs)
    table128 = _tc_pack_table(embed_table)
    parts = []
    for k in range(_NPART):
        idxk = inputs[:, k * _SPART:(k + 1) * _SPART]
        idxk = idxk.reshape(BATCH * _SPART // _S, _S)
        gk = _sc_gather(table128, idxk)
        pk = jnp.asarray(_POS[k * _SPART:(k + 1) * _SPART])
        parts.append(_tc_finish(gk, pk, _SPART))
    out = jnp.concatenate(parts, axis=1)
    return (out, mask.reshape(BATCH, 1, MAX_SEQ_LEN))


# speedup vs baseline: 1.0264x; 1.0264x over previous
import numpy as np
import jax
import jax.numpy as jnp
from jax import lax
from jax.experimental import pallas as pl
from jax.experimental.pallas import tpu as pltpu
from jax.experimental.pallas import tpu_sc as plsc

MAX_SEQ_LEN = 200
EMBED_DIM = 64
BATCH = 4096
VOCAB = 100000

_N = BATCH * MAX_SEQ_LEN
_S = 128
_NCH = _N // _S
_NC, _NS = 2, 16
_NW = _NC * _NS
_CPW = _NCH // _NW


def _pos_encoding(max_seq_len, wordvec_size):
    pos = np.arange(max_seq_len).reshape(1, -1).T
    i = np.arange(wordvec_size / 2).reshape(1, -1)
    pos_emb = np.empty((max_seq_len, wordvec_size))
    pos_emb[:, 0::2] = np.sin(pos / np.power(10000, 2 * i / wordvec_size))
    pos_emb[:, 1::2] = np.cos(pos / np.power(10000, 2 * i / wordvec_size))
    return pos_emb.astype(np.float32)


_POS = _pos_encoding(MAX_SEQ_LEN, EMBED_DIM)

_TC_PARAMS = pltpu.CompilerParams(dimension_semantics=("parallel",))


def _mask_body(x_ref, o_ref):
    o_ref[...] = (x_ref[...] != 0).astype(jnp.float32)


def _padding_mask(inputs):
    return pl.pallas_call(
        _mask_body,
        out_shape=jax.ShapeDtypeStruct((BATCH, MAX_SEQ_LEN), jnp.float32),
    )(inputs)


def _pack_body(t_ref, o_ref):
    t = t_ref[...]
    o_ref[...] = jnp.concatenate(
        [t, jnp.zeros((t.shape[0], 128 - EMBED_DIM), jnp.float32)], axis=1)


def _tc_pack_table(table):
    blk = 5000
    return pl.pallas_call(
        _pack_body,
        grid=(VOCAB // blk,),
        in_specs=[pl.BlockSpec((blk, EMBED_DIM), lambda i: (i, 0))],
        out_specs=pl.BlockSpec((blk, 128), lambda i: (i, 0)),
        out_shape=jax.ShapeDtypeStruct((VOCAB, 128), jnp.float32),
        compiler_params=_TC_PARAMS,
    )(table)


def _finish_body(g_ref, p_ref, o_ref):
    nb = o_ref.shape[0]
    x = g_ref[...].reshape(nb, MAX_SEQ_LEN, 128)
    o_ref[...] = x[:, :, 0:EMBED_DIM] + p_ref[...][None]


def _tc_finish(gathered, pos, nbatch):
    blk = 64
    return pl.pallas_call(
        _finish_body,
        grid=(nbatch // blk,),
        in_specs=[
            pl.BlockSpec((blk * MAX_SEQ_LEN, 128), lambda i: (i, 0)),
            pl.BlockSpec((MAX_SEQ_LEN, EMBED_DIM), lambda i: (0, 0)),
        ],
        out_specs=pl.BlockSpec((blk, MAX_SEQ_LEN, EMBED_DIM),
                               lambda i: (i, 0, 0)),
        out_shape=jax.ShapeDtypeStruct((nbatch, MAX_SEQ_LEN, EMBED_DIM),
                                       jnp.float32),
        compiler_params=_TC_PARAMS,
    )(gathered, pos)


def _sc_gather(table128, idx128):
    nch = idx128.shape[0]
    cpw = nch // _NW
    mesh = plsc.VectorSubcoreMesh(core_axis_name="c", subcore_axis_name="s")

    @pl.kernel(
        out_type=jax.ShapeDtypeStruct((nch * _S, 128), jnp.float32),
        mesh=mesh,
        compiler_params=pltpu.CompilerParams(use_tc_tiling_on_sc=True),
        scratch_types=[
            pltpu.VMEM((cpw + max((w * cpw) % 8 for w in range(_NW)),
                        _S), jnp.int32),
            pltpu.VMEM((4, _S, 128), jnp.float32),
            pltpu.SemaphoreType.DMA,
            pltpu.SemaphoreType.DMA,
            pltpu.SemaphoreType.DMA,
            pltpu.SemaphoreType.DMA,
            pltpu.SemaphoreType.DMA,
            pltpu.SemaphoreType.DMA,
            pltpu.SemaphoreType.DMA,
            pltpu.SemaphoreType.DMA,
        ],
    )
    def k(table_hbm, idx_hbm, out_hbm, idx_v, rows_v,
          sg0, sg1, sg2, sg3, so0, so1, so2, so3):
        wid = lax.axis_index("s") * _NC + lax.axis_index("c")
        base = wid * cpw
        s_g = (sg0, sg1, sg2, sg3)
        s_o = (so0, so1, so2, so3)

        maxoff = max((w * cpw) % 8 for w in range(_NW))
        assert ((_NW - 1) * cpw) % 8 == maxoff
        if maxoff:
            off = lax.rem(base, 8)
            abase = pl.multiple_of(base - off, 8)
            pltpu.sync_copy(idx_hbm.at[pl.ds(abase, cpw + maxoff)], idx_v)
        else:
            off = 0
            pltpu.sync_copy(idx_hbm.at[pl.ds(base, cpw)], idx_v)

        def gather(c, b):
            return pltpu.make_async_copy(
                table_hbm.at[idx_v.at[off + c]], rows_v.at[b], s_g[b])

        def out_copy(c, b):
            return pltpu.make_async_copy(
                rows_v.at[b],
                out_hbm.at[pl.ds((base + c) * _S, _S)], s_o[b])

        gather(0, 0).start()
        gather(1, 1).start()

        @pl.loop(0, cpw, step=4)
        def _(cc):
            for j in range(4):
                c = cc + j
                b = j
                nb = (j + 2) % 4

                gather(c, b).wait()
                out_copy(c, b).start()

                @pl.when(c >= 2)
                def _():
                    out_copy(c - 2, nb).wait()

                @pl.when(c + 2 < cpw)
                def _():
                    gather(c + 2, nb).start()

        out_copy(cpw - 2, 2).wait()
        out_copy(cpw - 1, 3).wait()

    return k(table128, idx128)


_NPART = 2
_BPART = BATCH // _NPART
_CHPART = _NCH // _NPART


def kernel(inputs, embed_table):
    mask = _padding_mask(inputs)
    table128 = _tc_pack_table(embed_table)
    idx128 = inputs.reshape(_NCH, _S)
    pos = jnp.asarray(_POS)
    parts = []
    for k in range(_NPART):
        gk = _sc_gather(table128, idx128[k * _CHPART:(k + 1) * _CHPART])
        parts.append(_tc_finish(gk, pos, _BPART))
    out = jnp.concatenate(parts, axis=0)
    return (out, mask.reshape(BATCH, 1, MAX_SEQ_LEN))


# speedup vs baseline: 1.1997x; 1.1688x over previous
import numpy as np
import jax
import jax.numpy as jnp
from jax import lax
from jax.experimental import pallas as pl
from jax.experimental.pallas import tpu as pltpu
from jax.experimental.pallas import tpu_sc as plsc

MAX_SEQ_LEN = 200
EMBED_DIM = 64
BATCH = 4096
VOCAB = 100000

_N = BATCH * MAX_SEQ_LEN
_S = 128
_NCH = _N // _S
_NC, _NS = 2, 16
_NW = _NC * _NS
_CPW = _NCH // _NW


def _pos_encoding(max_seq_len, wordvec_size):
    pos = np.arange(max_seq_len).reshape(1, -1).T
    i = np.arange(wordvec_size / 2).reshape(1, -1)
    pos_emb = np.empty((max_seq_len, wordvec_size))
    pos_emb[:, 0::2] = np.sin(pos / np.power(10000, 2 * i / wordvec_size))
    pos_emb[:, 1::2] = np.cos(pos / np.power(10000, 2 * i / wordvec_size))
    return pos_emb.astype(np.float32)


_POS = _pos_encoding(MAX_SEQ_LEN, EMBED_DIM)

_TC_PARAMS = pltpu.CompilerParams(dimension_semantics=("parallel",))


def _mask_body(x_ref, o_ref):
    o_ref[...] = (x_ref[...] != 0).astype(jnp.float32)


def _padding_mask(inputs):
    return pl.pallas_call(
        _mask_body,
        out_shape=jax.ShapeDtypeStruct((BATCH, MAX_SEQ_LEN), jnp.float32),
    )(inputs)


def _pack_body(t_ref, o_ref):
    t = t_ref[...]
    o_ref[...] = jnp.concatenate(
        [t, jnp.zeros((t.shape[0], 128 - EMBED_DIM), jnp.float32)], axis=1)


def _tc_pack_table(table):
    blk = 5000
    return pl.pallas_call(
        _pack_body,
        grid=(VOCAB // blk,),
        in_specs=[pl.BlockSpec((blk, EMBED_DIM), lambda i: (i, 0))],
        out_specs=pl.BlockSpec((blk, 128), lambda i: (i, 0)),
        out_shape=jax.ShapeDtypeStruct((VOCAB, 128), jnp.float32),
        compiler_params=_TC_PARAMS,
    )(table)


def _finish_body(g_ref, p_ref, o_ref):
    nb = o_ref.shape[0]
    x = g_ref[...].reshape(nb, MAX_SEQ_LEN, 128)
    o_ref[...] = x[:, :, 0:EMBED_DIM] + p_ref[...][None]


def _tc_finish(gathered, pos):
    blk = 64
    return pl.pallas_call(
        _finish_body,
        grid=(BATCH // blk,),
        in_specs=[
            pl.BlockSpec((blk * MAX_SEQ_LEN, 128), lambda i: (i, 0)),
            pl.BlockSpec((MAX_SEQ_LEN, EMBED_DIM), lambda i: (0, 0)),
        ],
        out_specs=pl.BlockSpec((blk, MAX_SEQ_LEN, EMBED_DIM),
                               lambda i: (i, 0, 0)),
        out_shape=jax.ShapeDtypeStruct((BATCH, MAX_SEQ_LEN, EMBED_DIM),
                                       jnp.float32),
        compiler_params=_TC_PARAMS,
    )(gathered, pos)


def _sc_gather(table128, idx128):
    mesh = plsc.VectorSubcoreMesh(core_axis_name="c", subcore_axis_name="s")
    D, L = 5, 3

    @pl.kernel(
        out_type=jax.ShapeDtypeStruct((_N, 128), jnp.float32),
        mesh=mesh,
        compiler_params=pltpu.CompilerParams(use_tc_tiling_on_sc=True),
        scratch_types=[
            pltpu.VMEM((_CPW, _S), jnp.int32),
            pltpu.VMEM((D, _S, 128), jnp.float32),
        ] + [pltpu.SemaphoreType.DMA] * (2 * D),
    )
    def k(table_hbm, idx_hbm, out_hbm, idx_v, rows_v, *sems):
        wid = lax.axis_index("s") * _NC + lax.axis_index("c")
        base = wid * _CPW
        s_g = sems[:D]
        s_o = sems[D:]

        pltpu.sync_copy(idx_hbm.at[pl.ds(base, _CPW)], idx_v)

        def gather(c, b):
            return pltpu.make_async_copy(
                table_hbm.at[idx_v.at[c]], rows_v.at[b], s_g[b])

        def out_copy(c, b):
            return pltpu.make_async_copy(
                rows_v.at[b],
                out_hbm.at[pl.ds((base + c) * _S, _S)], s_o[b])

        for c in range(L):
            gather(c, c).start()

        @pl.loop(0, _CPW, step=D)
        def _(cc):
            for j in range(D):
                c = cc + j

                gather(c, j).wait()
                out_copy(c, j).start()

                @pl.when(c >= D - L)
                def _():
                    out_copy(c - (D - L), (j + L) % D).wait()

                @pl.when(c + L < _CPW)
                def _():
                    gather(c + L, (j + L) % D).start()

        for i in range(D - L):
            out_copy(_CPW - (D - L) + i, (_CPW - (D - L) + i) % D).wait()

    return k(table128, idx128)


def kernel(inputs, embed_table):
    mask = _padding_mask(inputs)
    table128 = _tc_pack_table(embed_table)
    idx128 = inputs.reshape(_NCH, _S)
    gathered = _sc_gather(table128, idx128)
    out = _tc_finish(gathered, jnp.asarray(_POS))
    return (out, mask.reshape(BATCH, 1, MAX_SEQ_LEN))
